# baseline (speedup 1.0000x reference)
import functools

import jax
import jax.numpy as jnp
from jax.experimental import pallas as pl

B, D, K, P = 8, 64, 512, 256


def _vq_kernel(x_ref, e_ref, et_ref, q_ref, enc_ref, loss_ref, perp_ref):
    e = e_ref[...]
    et = et_ref[...]
    e2 = jnp.sum(et * et, axis=1, keepdims=True)
    iota_k = jax.lax.broadcasted_iota(jnp.int32, (K, P), 0)
    inf = jnp.float32(jnp.inf)
    ones_col = jnp.ones((P, 1), jnp.float32)

    loss_sum = jnp.float32(0.0)
    counts = jnp.zeros((K, 1), jnp.float32)
    for bb in range(B):
        x = x_ref[bb]
        s = jax.lax.dot_general(et, x, (((1,), (0,)), ((), ())),
                                precision=jax.lax.Precision.HIGHEST)
        a = e2 - 2.0 * s

        m0 = jnp.min(a, axis=0, keepdims=True)
        i0 = jnp.min(jnp.where(a == m0, iota_k, K), axis=0, keepdims=True)
        a1m = jnp.where(iota_k == i0, inf, a)
        m1 = jnp.min(a1m, axis=0, keepdims=True)
        i1 = jnp.min(jnp.where(a1m == m1, iota_k, K), axis=0, keepdims=True)

        oh0 = (iota_k == i0).astype(jnp.float32)
        oh1 = (iota_k == i1).astype(jnp.float32)
        q0 = jax.lax.dot(e, oh0, precision=jax.lax.Precision.HIGHEST)
        q1 = jax.lax.dot(e, oh1, precision=jax.lax.Precision.HIGHEST)

        acc0 = jnp.zeros((1, P), jnp.float32)
        acc1 = jnp.zeros((1, P), jnp.float32)
        for d in range(D):
            xd = x[d:d + 1, :]
            d0 = q0[d:d + 1, :] - xd
            d1 = q1[d:d + 1, :] - xd
            acc0 = acc0 + d0 * d0
            acc1 = acc1 + d1 * d1
        s0 = jnp.sqrt(acc0)
        s1 = jnp.sqrt(acc1)

        w1 = (s1 < s0) | ((s1 == s0) & (i1 < i0))
        enc = jnp.where(w1, oh1, oh0)
        enc_ref[bb] = enc
        q = jnp.where(w1, q1, q0)
        q_ref[bb] = x + (q - x)

        loss_sum = loss_sum + jnp.sum(jnp.where(w1, acc1, acc0))
        counts = counts + jax.lax.dot(
            enc, ones_col, precision=jax.lax.Precision.HIGHEST)

    loss_ref[...] = jnp.full((1, 1), loss_sum / (B * D * P), jnp.float32)
    avg = counts / (B * P)
    ent = jnp.sum(avg * jnp.log(avg + 1e-10))
    perp_ref[...] = jnp.full((1, 1), jnp.exp(-ent) / K, jnp.float32)


@functools.partial(jax.jit, static_argnames=())
def _vq_call(x, e, et):
    return pl.pallas_call(
        _vq_kernel,
        out_shape=[
            jax.ShapeDtypeStruct((B, D, P), jnp.float32),
            jax.ShapeDtypeStruct((B, K, P), jnp.float32),
            jax.ShapeDtypeStruct((1, 1), jnp.float32),
            jax.ShapeDtypeStruct((1, 1), jnp.float32),
        ],
    )(x, e, et)


def kernel(input, embedding):
    b, d, h, w = input.shape
    x = input.reshape(b, d, h * w)
    e = embedding[:, :, 0]
    et = e.T
    q, enc, loss, perp = _vq_call(x, e, et)
    return (q.reshape(b, d, h, w),
            enc.reshape(b, K, h, w),
            loss.reshape(()),
            perp.reshape(1))


# speedup vs baseline: 1.4129x; 1.4129x over previous
import functools

import jax
import jax.numpy as jnp
from jax.experimental import pallas as pl

B, D, K, P = 8, 64, 512, 256
BP = B * P

_DEF = jax.lax.Precision.DEFAULT


def _split3(m):
    hi = jnp.asarray(m, jnp.bfloat16).astype(jnp.float32)
    r = m - hi
    mid = jnp.asarray(r, jnp.bfloat16).astype(jnp.float32)
    lo = r - mid
    return hi, mid, lo


def _vq_kernel(x_ref, e_ref, et_ref, q_ref, enc_ref, loss_ref, perp_ref):
    e = e_ref[...]
    et = et_ref[...]
    x3 = x_ref[...]
    xx = jnp.concatenate([x3[bb] for bb in range(B)], axis=1)

    et_hi, et_mid, et_lo = _split3(et)
    x_hi = jnp.asarray(xx, jnp.bfloat16).astype(jnp.float32)
    x_lo = xx - x_hi
    dn = (((1,), (0,)), ((), ()))
    s = (jax.lax.dot_general(et_hi, x_hi, dn, precision=_DEF)
         + (jax.lax.dot_general(et_hi, x_lo, dn, precision=_DEF)
            + jax.lax.dot_general(et_mid, x_hi, dn, precision=_DEF)))
    e2 = jnp.sum(et * et, axis=1, keepdims=True)
    a = e2 - 2.0 * s

    iota_k = jax.lax.broadcasted_iota(jnp.int32, (K, BP), 0)
    inf = jnp.float32(jnp.inf)
    i0 = jnp.argmin(a, axis=0, keepdims=True)
    a1m = jnp.where(iota_k == i0, inf, a)
    i1 = jnp.argmin(a1m, axis=0, keepdims=True)

    oh0 = (iota_k == i0).astype(jnp.float32)
    oh1 = (iota_k == i1).astype(jnp.float32)
    oh = jnp.concatenate([oh0, oh1], axis=1)

    e_hi, e_mid, e_lo = _split3(e)
    qq = (jax.lax.dot(e_hi, oh, precision=_DEF)
          + jax.lax.dot(e_mid, oh, precision=_DEF)
          + jax.lax.dot(e_lo, oh, precision=_DEF))

    x2 = jnp.concatenate([xx, xx], axis=1)
    acc = jnp.zeros((1, 2 * BP), jnp.float32)
    for d in range(D):
        dd = qq[d:d + 1, :] - x2[d:d + 1, :]
        acc = acc + dd * dd
    sq = jnp.sqrt(acc)
    s0 = sq[:, :BP]
    s1 = sq[:, BP:]

    w1 = (s1 < s0) | ((s1 == s0) & (i1 < i0))
    enc = jnp.where(w1, oh[:, BP:], oh[:, :BP])
    qw = jnp.where(w1, qq[:, BP:], qq[:, :BP])
    st = xx + (qw - xx)
    for bb in range(B):
        sl = slice(bb * P, (bb + 1) * P)
        enc_ref[bb] = enc[:, sl]
        q_ref[bb] = st[:, sl]

    loss_sum = jnp.sum(jnp.where(w1, acc[:, BP:], acc[:, :BP]))
    loss_ref[...] = jnp.full((1, 1), loss_sum / (B * D * P), jnp.float32)
    counts = jnp.sum(enc, axis=1, keepdims=True)
    avg = counts / (B * P)
    ent = jnp.sum(avg * jnp.log(avg + 1e-10))
    perp_ref[...] = jnp.full((1, 1), jnp.exp(-ent) / K, jnp.float32)


@functools.partial(jax.jit, static_argnames=())
def _vq_call(x, e, et):
    return pl.pallas_call(
        _vq_kernel,
        out_shape=[
            jax.ShapeDtypeStruct((B, D, P), jnp.float32),
            jax.ShapeDtypeStruct((B, K, P), jnp.float32),
            jax.ShapeDtypeStruct((1, 1), jnp.float32),
            jax.ShapeDtypeStruct((1, 1), jnp.float32),
        ],
    )(x, e, et)


def kernel(input, embedding):
    b, d, h, w = input.shape
    x = input.reshape(b, d, h * w)
    e = embedding[:, :, 0]
    et = e.T
    q, enc, loss, perp = _vq_call(x, e, et)
    return (q.reshape(b, d, h, w),
            enc.reshape(b, K, h, w),
            loss.reshape(()),
            perp.reshape(1))
